# baseline (speedup 1.0000x reference)
import functools

import numpy as np

import jax
import jax.numpy as jnp
from jax import lax
from jax.experimental import pallas as pl
from jax.experimental.pallas import tpu as pltpu
from jax.experimental.pallas import tpu_sc as plsc

NUM_EXPERT = 16
D_MODEL = 2048
T_TOKENS = 8192
CAP = T_TOKENS // NUM_EXPERT
ROW_BLK = 1024
L = 16
STEPS = T_TOKENS // L
NBUCKET = 256
MININT = np.int32(-(2**31))
NEG1 = np.int32(-1)


def _affin_body(c_ref, x_ref, o_ref):
    o_ref[...] = jax.lax.dot_general(
        c_ref[...], x_ref[...],
        dimension_numbers=(((1,), (1,)), ((), ())),
        preferred_element_type=jnp.float32,
    )


def _affinities_t(centroids, features):
    return pl.pallas_call(
        _affin_body,
        grid=(T_TOKENS // ROW_BLK,),
        in_specs=[
            pl.BlockSpec((NUM_EXPERT, D_MODEL), lambda i: (0, 0)),
            pl.BlockSpec((ROW_BLK, D_MODEL), lambda i: (i, 0)),
        ],
        out_specs=pl.BlockSpec((NUM_EXPERT, ROW_BLK), lambda i: (0, i)),
        out_shape=jax.ShapeDtypeStruct((NUM_EXPERT, T_TOKENS), jnp.float32),
    )(centroids, features)


def _assign_body(affin_hbm, idx_out, val_out, col_raw, key0, key1, pay0, pay1,
                 hist, totbuf, assigned, cbuf, row_i, fbuf, pbuf,
                 tbuf, ebuf, obuf, vrow, s_idx, s_out):
    cid = lax.axis_index("c")
    sid = lax.axis_index("s")
    lane = lax.iota(jnp.int32, L)
    on_core0 = cid == 0
    ones = jnp.ones((L,), jnp.int32)
    zeros = jnp.zeros((L,), jnp.int32)
    lane_eq15 = lane == 15

    UNROLL = 4

    @pl.when(on_core0)
    def _sort():
      with jax.named_scope("phase_sort"):
        w = sid
        pltpu.sync_copy(affin_hbm.at[w], col_raw)

        def kprep(i, _):
            for u in range(UNROLL):
                o = (i * UNROLL + u) * L
                v = col_raw[pl.ds(o, L)]
                b = lax.bitcast_convert_type(v, jnp.int32)
                m = b >> 31
                asc = b ^ (m | MININT)
                key0[pl.ds(o + (o >> 9), L)] = asc ^ NEG1
            return 0

        lax.fori_loop(0, STEPS // UNROLL, kprep, 0)

        lane_base = lane * (STEPS + 1)
        lane_logical = lane * STEPS

        def radix_pass(shift, srck, srcp, dstk, dstp, first, last=False):
            def zh(i, _):
                for u in range(UNROLL):
                    hist[pl.ds((i * UNROLL + u) * L, L)] = zeros
                return 0

            lax.fori_loop(0, NBUCKET // UNROLL, zh, 0)

            def ha(s, _):
                for u in range(UNROLL):
                    kk = plsc.load_gather(srck, [lane_base + (s * UNROLL + u)])
                    d = lax.shift_right_logical(kk, shift) & 255
                    plsc.addupdate_scatter(hist, [d * L + lane], ones)
                return 0

            lax.fori_loop(0, STEPS // UNROLL, ha, 0)

            def sc(i, carry):
                h = hist[pl.ds(i * L, L)]
                inc = plsc.cumsum(h)
                tot = jnp.sum(h)
                hist[pl.ds(i * L, L)] = inc - h + carry
                return carry + tot

            lax.fori_loop(0, NBUCKET, sc, jnp.int32(0))

            def pb(s, _):
                for u in range(UNROLL):
                    su = s * UNROLL + u
                    gidx = lane_base + su
                    kk = plsc.load_gather(srck, [gidx])
                    if first:
                        pay = lane_logical + su
                    else:
                        pay = plsc.load_gather(srcp, [gidx])
                    d = lax.shift_right_logical(kk, shift) & 255
                    addr = d * L + lane
                    pos = plsc.load_gather(hist, [addr])
                    if last:
                        ppos = pos
                    else:
                        ppos = pos + (pos >> 9)
                    plsc.store_scatter(dstk, [ppos], kk)
                    plsc.store_scatter(dstp, [ppos], pay)
                    plsc.store_scatter(hist, [addr], pos + 1)
                return 0

            lax.fori_loop(0, STEPS // UNROLL, pb, 0)

        radix_pass(0, key0, pay0, key1, pay1, True)
        radix_pass(8, key1, pay1, key0, pay0, False)
        radix_pass(16, key0, pay0, key1, pay1, False)
        radix_pass(24, key1, pay1, key0, pay0, False, last=True)
        pltpu.sync_copy(pay0.at[pl.ds(0, T_TOKENS)],
                        s_idx.at[pl.ds(sid * T_TOKENS, T_TOKENS)])

    plsc.subcore_barrier()

    CHUNK = 512
    CHUNK_V = CHUNK // L

    @pl.when(jnp.logical_and(on_core0, sid == 0))
    def _walk():
      with jax.named_scope("phase_walk"):
        def za(i, _):
            for u in range(UNROLL):
                assigned[pl.ds((i * UNROLL + u) * L, L)] = zeros
            return 0

        lax.fori_loop(0, STEPS // UNROLL, za, 0)

        def expert(e, _):
            pltpu.sync_copy(s_idx.at[pl.ds(e * T_TOKENS, T_TOKENS)], cbuf)

            def wcond(c):
                ci, cnt = c
                return jnp.logical_and(cnt < CAP, ci < NUM_EXPERT)

            def wbody(c):
                ci, cnt = c
                base = ci * CHUNK

                def c1(j, _):
                    for u in range(UNROLL):
                        jj = j * UNROLL + u
                        ids = cbuf[pl.ds(base + jj * L, L)]
                        fl = plsc.load_gather(assigned, [ids])
                        free = fl == 0
                        fbuf[pl.ds(jj * L, L)] = jnp.where(free, 1, 0)
                        t = plsc.all_reduce_population_count(free)
                        plsc.store_scatter(tbuf, [lane * 0 + jj], t,
                                           mask=lane_eq15)
                    return 0

                lax.fori_loop(0, CHUNK_V // UNROLL, c1, 0)

                t0 = tbuf[pl.ds(0, L)]
                t1 = tbuf[pl.ds(L, L)]
                s0 = jnp.sum(t0)
                ebuf[pl.ds(0, L)] = plsc.cumsum(t0) - t0
                ebuf[pl.ds(L, L)] = plsc.cumsum(t1) - t1 + s0
                ctot = s0 + jnp.sum(t1)

                def c3(j, _):
                    for u in range(UNROLL):
                        jj = j * UNROLL + u
                        ids = cbuf[pl.ds(base + jj * L, L)]
                        freei = fbuf[pl.ds(jj * L, L)]
                        pv = plsc.cumsum(freei)
                        eoff = plsc.load_gather(ebuf, [lane * 0 + jj])
                        pos = cnt + eoff + pv - 1
                        take = jnp.logical_and(freei == 1, pos < CAP)
                        plsc.store_scatter(row_i, [pos], ids, mask=take)
                        plsc.store_scatter(assigned, [ids], ones, mask=take)
                    return 0

                lax.fori_loop(0, CHUNK_V // UNROLL, c3, 0)
                return (ci + 1, jnp.minimum(cnt + ctot, CAP))

            lax.while_loop(wcond, wbody, (jnp.int32(0), jnp.int32(0)))
            pltpu.sync_copy(row_i, idx_out.at[e])
            pltpu.sync_copy(row_i, s_out.at[pl.ds(e * CAP, CAP)])
            return 0

        lax.fori_loop(0, NUM_EXPERT, expert, 0)

    plsc.subcore_barrier()

    @pl.when(on_core0)
    def _emit():
      with jax.named_scope("phase_emit"):
        pltpu.sync_copy(s_out.at[pl.ds(sid * CAP, CAP)], obuf)

        def ev(i, _):
            for u in range(UNROLL):
                o = (i * UNROLL + u) * L
                idsv = obuf[pl.ds(o, L)]
                vrow[pl.ds(o, L)] = plsc.load_gather(col_raw, [idsv])
            return 0

        lax.fori_loop(0, CAP // L // UNROLL, ev, 0)
        pltpu.sync_copy(vrow, val_out.at[sid])


def _assign_sc(affin_t):
    mesh = plsc.VectorSubcoreMesh(core_axis_name="c", subcore_axis_name="s")
    f = functools.partial(
        pl.kernel,
        mesh=mesh,
        compiler_params=pltpu.CompilerParams(needs_layout_passes=False),
        out_type=[
            jax.ShapeDtypeStruct((NUM_EXPERT, CAP), jnp.int32),
            jax.ShapeDtypeStruct((NUM_EXPERT, CAP), jnp.float32),
        ],
        scratch_types=[
            pltpu.VMEM((T_TOKENS,), jnp.float32),
            pltpu.VMEM((T_TOKENS + L,), jnp.int32),
            pltpu.VMEM((T_TOKENS + L,), jnp.int32),
            pltpu.VMEM((T_TOKENS + L,), jnp.int32),
            pltpu.VMEM((T_TOKENS + L,), jnp.int32),
            pltpu.VMEM((NBUCKET * L,), jnp.int32),
            pltpu.VMEM((NBUCKET + L,), jnp.int32),
            pltpu.VMEM((T_TOKENS,), jnp.int32),
            pltpu.VMEM((T_TOKENS,), jnp.int32),
            pltpu.VMEM((CAP,), jnp.int32),
            pltpu.VMEM((512,), jnp.int32),
            pltpu.VMEM((512,), jnp.int32),
            pltpu.VMEM((32,), jnp.int32),
            pltpu.VMEM((32 + L,), jnp.int32),
            pltpu.VMEM((CAP,), jnp.int32),
            pltpu.VMEM((CAP,), jnp.float32),
            pltpu.VMEM_SHARED((NUM_EXPERT * T_TOKENS,), jnp.int32),
            pltpu.VMEM_SHARED((NUM_EXPERT * CAP,), jnp.int32),
        ],
    )(_assign_body)
    return f(affin_t)


def kernel(input_features, expert_centroids):
    features = input_features.reshape(-1, input_features.shape[-1])
    affin_t = _affinities_t(expert_centroids, features)
    top_idx, top_value = _assign_sc(affin_t)
    return top_idx, top_value


# speedup vs baseline: 1.0362x; 1.0362x over previous
import functools

import numpy as np

import jax
import jax.numpy as jnp
from jax import lax
from jax.experimental import pallas as pl
from jax.experimental.pallas import tpu as pltpu
from jax.experimental.pallas import tpu_sc as plsc

NUM_EXPERT = 16
D_MODEL = 2048
T_TOKENS = 8192
CAP = T_TOKENS // NUM_EXPERT
ROW_BLK = 1024
L = 16
STEPS = T_TOKENS // L
NBUCKET = 256
MININT = np.int32(-(2**31))
NEG1 = np.int32(-1)


def _affin_body(c_ref, x_ref, o_ref):
    o_ref[...] = jax.lax.dot_general(
        c_ref[...], x_ref[...],
        dimension_numbers=(((1,), (1,)), ((), ())),
        preferred_element_type=jnp.float32,
    )


def _affinities_t(centroids, features):
    return pl.pallas_call(
        _affin_body,
        grid=(T_TOKENS // ROW_BLK,),
        in_specs=[
            pl.BlockSpec((NUM_EXPERT, D_MODEL), lambda i: (0, 0)),
            pl.BlockSpec((ROW_BLK, D_MODEL), lambda i: (i, 0)),
        ],
        out_specs=pl.BlockSpec((NUM_EXPERT, ROW_BLK), lambda i: (0, i)),
        out_shape=jax.ShapeDtypeStruct((NUM_EXPERT, T_TOKENS), jnp.float32),
    )(centroids, features)


def _assign_body(affin_hbm, idx_out, val_out, col_raw, key0, key1, pay0, pay1,
                 hist, totbuf, assigned, cbuf, cbuf2, row_i, fbuf, pbuf,
                 tbuf, ebuf, obuf, vrow, semA, semB, s_idx, s_out):
    cid = lax.axis_index("c")
    sid = lax.axis_index("s")
    lane = lax.iota(jnp.int32, L)
    on_core0 = cid == 0
    ones = jnp.ones((L,), jnp.int32)
    zeros = jnp.zeros((L,), jnp.int32)
    lane_eq15 = lane == 15

    UNROLL = 4

    @pl.when(on_core0)
    def _sort():
      with jax.named_scope("phase_sort"):
        w = sid
        pltpu.sync_copy(affin_hbm.at[w], col_raw)

        def kprep(i, _):
            for u in range(UNROLL):
                o = (i * UNROLL + u) * L
                v = col_raw[pl.ds(o, L)]
                b = lax.bitcast_convert_type(v, jnp.int32)
                m = b >> 31
                asc = b ^ (m | MININT)
                key0[pl.ds(o + (o >> 9), L)] = asc ^ NEG1
            return 0

        lax.fori_loop(0, STEPS // UNROLL, kprep, 0)

        lane_base = lane * (STEPS + 1)
        lane_logical = lane * STEPS

        def radix_pass(shift, srck, srcp, dstk, dstp, first, last=False):
            def zh(i, _):
                for u in range(UNROLL):
                    hist[pl.ds((i * UNROLL + u) * L, L)] = zeros
                return 0

            lax.fori_loop(0, NBUCKET // UNROLL, zh, 0)

            def ha(s, _):
                for u in range(UNROLL):
                    kk = plsc.load_gather(srck, [lane_base + (s * UNROLL + u)])
                    d = lax.shift_right_logical(kk, shift) & 255
                    plsc.addupdate_scatter(hist, [d * L + lane], ones)
                return 0

            lax.fori_loop(0, STEPS // UNROLL, ha, 0)

            def sc(i, carry):
                h = hist[pl.ds(i * L, L)]
                inc = plsc.cumsum(h)
                tot = jnp.sum(h)
                hist[pl.ds(i * L, L)] = inc - h + carry
                return carry + tot

            lax.fori_loop(0, NBUCKET, sc, jnp.int32(0))

            def pb(s, _):
                for u in range(UNROLL):
                    su = s * UNROLL + u
                    gidx = lane_base + su
                    kk = plsc.load_gather(srck, [gidx])
                    if first:
                        pay = lane_logical + su
                    else:
                        pay = plsc.load_gather(srcp, [gidx])
                    d = lax.shift_right_logical(kk, shift) & 255
                    addr = d * L + lane
                    pos = plsc.load_gather(hist, [addr])
                    if last:
                        ppos = pos
                    else:
                        ppos = pos + (pos >> 9)
                    plsc.store_scatter(dstk, [ppos], kk)
                    plsc.store_scatter(dstp, [ppos], pay)
                    plsc.store_scatter(hist, [addr], pos + 1)
                return 0

            lax.fori_loop(0, STEPS // UNROLL, pb, 0)

        radix_pass(0, key0, pay0, key1, pay1, True)
        radix_pass(8, key1, pay1, key0, pay0, False)
        radix_pass(16, key0, pay0, key1, pay1, False)
        radix_pass(24, key1, pay1, key0, pay0, False, last=True)
        pltpu.sync_copy(pay0.at[pl.ds(0, T_TOKENS)],
                        s_idx.at[pl.ds(sid * T_TOKENS, T_TOKENS)])

    plsc.subcore_barrier()

    CHUNK = 512
    CHUNK_V = CHUNK // L

    @pl.when(jnp.logical_and(on_core0, sid == 0))
    def _walk():
      with jax.named_scope("phase_walk"):
        def za(i, _):
            for u in range(UNROLL):
                assigned[pl.ds((i * UNROLL + u) * L, L)] = zeros
            return 0

        lax.fori_loop(0, STEPS // UNROLL, za, 0)

        def process(e, buf):
            def wcond(c):
                ci, cnt = c
                return jnp.logical_and(cnt < CAP, ci < NUM_EXPERT)

            def wbody(c):
                ci, cnt = c
                base = ci * CHUNK

                def c1(j, _):
                    for u in range(UNROLL):
                        jj = j * UNROLL + u
                        ids = buf[pl.ds(base + jj * L, L)]
                        fl = plsc.load_gather(assigned, [ids])
                        free = fl == 0
                        fbuf[pl.ds(jj * L, L)] = jnp.where(free, 1, 0)
                        t = plsc.all_reduce_population_count(free)
                        plsc.store_scatter(tbuf, [lane * 0 + jj], t,
                                           mask=lane_eq15)
                    return 0

                lax.fori_loop(0, CHUNK_V // UNROLL, c1, 0)

                t0 = tbuf[pl.ds(0, L)]
                t1 = tbuf[pl.ds(L, L)]
                s0 = jnp.sum(t0)
                ebuf[pl.ds(0, L)] = plsc.cumsum(t0) - t0
                ebuf[pl.ds(L, L)] = plsc.cumsum(t1) - t1 + s0
                ctot = s0 + jnp.sum(t1)

                def c3(j, _):
                    for u in range(UNROLL):
                        jj = j * UNROLL + u
                        ids = buf[pl.ds(base + jj * L, L)]
                        freei = fbuf[pl.ds(jj * L, L)]
                        pv = plsc.cumsum(freei)
                        eoff = plsc.load_gather(ebuf, [lane * 0 + jj])
                        pos = cnt + eoff + pv - 1
                        take = jnp.logical_and(freei == 1, pos < CAP)
                        plsc.store_scatter(row_i, [pos], ids, mask=take)
                        plsc.store_scatter(assigned, [ids], ones, mask=take)
                    return 0

                lax.fori_loop(0, CHUNK_V // UNROLL, c3, 0)
                return (ci + 1, jnp.minimum(cnt + ctot, CAP))

            lax.while_loop(wcond, wbody, (jnp.int32(0), jnp.int32(0)))
            pltpu.sync_copy(row_i, idx_out.at[e])
            pltpu.sync_copy(row_i, s_out.at[pl.ds(e * CAP, CAP)])

        pltpu.async_copy(s_idx.at[pl.ds(0, T_TOKENS)], cbuf, semA)

        def expert2(i, _):
            e0 = 2 * i
            pltpu.make_async_copy(
                s_idx.at[pl.ds(0, T_TOKENS)], cbuf, semA).wait()
            pltpu.async_copy(
                s_idx.at[pl.ds((e0 + 1) * T_TOKENS, T_TOKENS)], cbuf2, semB)
            process(e0, cbuf)

            pltpu.make_async_copy(
                s_idx.at[pl.ds(0, T_TOKENS)], cbuf2, semB).wait()

            @pl.when(i < NUM_EXPERT // 2 - 1)
            def _pf():
                pltpu.async_copy(
                    s_idx.at[pl.ds((e0 + 2) * T_TOKENS, T_TOKENS)], cbuf,
                    semA)

            process(e0 + 1, cbuf2)
            return 0

        lax.fori_loop(0, NUM_EXPERT // 2, expert2, 0)

    plsc.subcore_barrier()

    @pl.when(on_core0)
    def _emit():
      with jax.named_scope("phase_emit"):
        pltpu.sync_copy(s_out.at[pl.ds(sid * CAP, CAP)], obuf)

        def ev(i, _):
            for u in range(UNROLL):
                o = (i * UNROLL + u) * L
                idsv = obuf[pl.ds(o, L)]
                vrow[pl.ds(o, L)] = plsc.load_gather(col_raw, [idsv])
            return 0

        lax.fori_loop(0, CAP // L // UNROLL, ev, 0)
        pltpu.sync_copy(vrow, val_out.at[sid])


def _assign_sc(affin_t):
    mesh = plsc.VectorSubcoreMesh(core_axis_name="c", subcore_axis_name="s")
    f = functools.partial(
        pl.kernel,
        mesh=mesh,
        compiler_params=pltpu.CompilerParams(needs_layout_passes=False),
        out_type=[
            jax.ShapeDtypeStruct((NUM_EXPERT, CAP), jnp.int32),
            jax.ShapeDtypeStruct((NUM_EXPERT, CAP), jnp.float32),
        ],
        scratch_types=[
            pltpu.VMEM((T_TOKENS,), jnp.float32),
            pltpu.VMEM((T_TOKENS + L,), jnp.int32),
            pltpu.VMEM((T_TOKENS + L,), jnp.int32),
            pltpu.VMEM((T_TOKENS + L,), jnp.int32),
            pltpu.VMEM((T_TOKENS + L,), jnp.int32),
            pltpu.VMEM((NBUCKET * L,), jnp.int32),
            pltpu.VMEM((NBUCKET + L,), jnp.int32),
            pltpu.VMEM((T_TOKENS,), jnp.int32),
            pltpu.VMEM((T_TOKENS,), jnp.int32),
            pltpu.VMEM((T_TOKENS,), jnp.int32),
            pltpu.VMEM((CAP,), jnp.int32),
            pltpu.VMEM((512,), jnp.int32),
            pltpu.VMEM((512,), jnp.int32),
            pltpu.VMEM((32,), jnp.int32),
            pltpu.VMEM((32 + L,), jnp.int32),
            pltpu.VMEM((CAP,), jnp.int32),
            pltpu.VMEM((CAP,), jnp.float32),
            pltpu.SemaphoreType.DMA,
            pltpu.SemaphoreType.DMA,
            pltpu.VMEM_SHARED((NUM_EXPERT * T_TOKENS,), jnp.int32),
            pltpu.VMEM_SHARED((NUM_EXPERT * CAP,), jnp.int32),
        ],
    )(_assign_body)
    return f(affin_t)


def kernel(input_features, expert_centroids):
    features = input_features.reshape(-1, input_features.shape[-1])
    affin_t = _affinities_t(expert_centroids, features)
    top_idx, top_value = _assign_sc(affin_t)
    return top_idx, top_value


# speedup vs baseline: 1.0453x; 1.0088x over previous
import functools

import numpy as np

import jax
import jax.numpy as jnp
from jax import lax
from jax.experimental import pallas as pl
from jax.experimental.pallas import tpu as pltpu
from jax.experimental.pallas import tpu_sc as plsc

NUM_EXPERT = 16
D_MODEL = 2048
T_TOKENS = 8192
CAP = T_TOKENS // NUM_EXPERT
ROW_BLK = 1024
L = 16
STEPS = T_TOKENS // L
NBUCKET = 256
MININT = np.int32(-(2**31))
NEG1 = np.int32(-1)


def _affin_body(c_ref, x_ref, o_ref):
    o_ref[...] = jax.lax.dot_general(
        c_ref[...], x_ref[...],
        dimension_numbers=(((1,), (1,)), ((), ())),
        preferred_element_type=jnp.float32,
    )


def _affinities_t(centroids, features):
    return pl.pallas_call(
        _affin_body,
        grid=(T_TOKENS // ROW_BLK,),
        in_specs=[
            pl.BlockSpec((NUM_EXPERT, D_MODEL), lambda i: (0, 0)),
            pl.BlockSpec((ROW_BLK, D_MODEL), lambda i: (i, 0)),
        ],
        out_specs=pl.BlockSpec((NUM_EXPERT, ROW_BLK), lambda i: (0, i)),
        out_shape=jax.ShapeDtypeStruct((NUM_EXPERT, T_TOKENS), jnp.float32),
    )(centroids, features)


def _assign_body(affin_hbm, idx_out, val_out, col_raw, key0, key1, pay0, pay1,
                 hist, totbuf, assigned, cbuf, cbuf2, row_i, fbuf, pbuf,
                 tbuf, ebuf, obuf, vrow, semA, semB, s_idx, s_out):
    cid = lax.axis_index("c")
    sid = lax.axis_index("s")
    lane = lax.iota(jnp.int32, L)
    on_core0 = cid == 0
    ones = jnp.ones((L,), jnp.int32)
    zeros = jnp.zeros((L,), jnp.int32)
    lane_eq15 = lane == 15

    UNROLL = 4
    UNROLL2 = 8

    @pl.when(on_core0)
    def _sort():
      with jax.named_scope("phase_sort"):
        w = sid
        pltpu.sync_copy(affin_hbm.at[w], col_raw)

        def kprep(i, _):
            for u in range(UNROLL):
                o = (i * UNROLL + u) * L
                v = col_raw[pl.ds(o, L)]
                b = lax.bitcast_convert_type(v, jnp.int32)
                m = b >> 31
                asc = b ^ (m | MININT)
                key0[pl.ds(o + (o >> 9), L)] = asc ^ NEG1
            return 0

        lax.fori_loop(0, STEPS // UNROLL, kprep, 0)

        lane_base = lane * (STEPS + 1)
        lane_logical = lane * STEPS

        def radix_pass(shift, srck, srcp, dstk, dstp, first, last=False):
            def zh(i, _):
                for u in range(UNROLL):
                    hist[pl.ds((i * UNROLL + u) * L, L)] = zeros
                return 0

            lax.fori_loop(0, NBUCKET // UNROLL, zh, 0)

            def ha(s, _):
                for u in range(UNROLL2):
                    kk = plsc.load_gather(srck, [lane_base + (s * UNROLL2 + u)])
                    d = lax.shift_right_logical(kk, shift) & 255
                    plsc.addupdate_scatter(hist, [d * L + lane], ones)
                return 0

            lax.fori_loop(0, STEPS // UNROLL2, ha, 0)

            def sc(i, carry):
                h = hist[pl.ds(i * L, L)]
                inc = plsc.cumsum(h)
                tot = jnp.sum(h)
                hist[pl.ds(i * L, L)] = inc - h + carry
                return carry + tot

            lax.fori_loop(0, NBUCKET, sc, jnp.int32(0))

            def pb(s, _):
                for u in range(UNROLL2):
                    su = s * UNROLL2 + u
                    gidx = lane_base + su
                    kk = plsc.load_gather(srck, [gidx])
                    if first:
                        pay = lane_logical + su
                    else:
                        pay = plsc.load_gather(srcp, [gidx])
                    d = lax.shift_right_logical(kk, shift) & 255
                    addr = d * L + lane
                    pos = plsc.load_gather(hist, [addr])
                    if last:
                        ppos = pos
                    else:
                        ppos = pos + (pos >> 9)
                    plsc.store_scatter(dstk, [ppos], kk)
                    plsc.store_scatter(dstp, [ppos], pay)
                    plsc.store_scatter(hist, [addr], pos + 1)
                return 0

            lax.fori_loop(0, STEPS // UNROLL2, pb, 0)

        radix_pass(0, key0, pay0, key1, pay1, True)
        radix_pass(8, key1, pay1, key0, pay0, False)
        radix_pass(16, key0, pay0, key1, pay1, False)
        radix_pass(24, key1, pay1, key0, pay0, False, last=True)
        pltpu.sync_copy(pay0.at[pl.ds(0, T_TOKENS)],
                        s_idx.at[pl.ds(sid * T_TOKENS, T_TOKENS)])

    plsc.subcore_barrier()

    CHUNK = 512
    CHUNK_V = CHUNK // L

    @pl.when(jnp.logical_and(on_core0, sid == 0))
    def _walk():
      with jax.named_scope("phase_walk"):
        def za(i, _):
            for u in range(UNROLL):
                assigned[pl.ds((i * UNROLL + u) * L, L)] = zeros
            return 0

        lax.fori_loop(0, STEPS // UNROLL, za, 0)

        def process(e, buf):
            def wcond(c):
                ci, cnt = c
                return jnp.logical_and(cnt < CAP, ci < NUM_EXPERT)

            def wbody(c):
                ci, cnt = c
                base = ci * CHUNK

                def c1(j, _):
                    for u in range(UNROLL):
                        jj = j * UNROLL + u
                        ids = buf[pl.ds(base + jj * L, L)]
                        fl = plsc.load_gather(assigned, [ids])
                        free = fl == 0
                        fbuf[pl.ds(jj * L, L)] = jnp.where(free, 1, 0)
                        t = plsc.all_reduce_population_count(free)
                        plsc.store_scatter(tbuf, [lane * 0 + jj], t,
                                           mask=lane_eq15)
                    return 0

                lax.fori_loop(0, CHUNK_V // UNROLL, c1, 0)

                t0 = tbuf[pl.ds(0, L)]
                t1 = tbuf[pl.ds(L, L)]
                s0 = jnp.sum(t0)
                ebuf[pl.ds(0, L)] = plsc.cumsum(t0) - t0
                ebuf[pl.ds(L, L)] = plsc.cumsum(t1) - t1 + s0
                ctot = s0 + jnp.sum(t1)

                def c3(j, _):
                    for u in range(UNROLL):
                        jj = j * UNROLL + u
                        ids = buf[pl.ds(base + jj * L, L)]
                        freei = fbuf[pl.ds(jj * L, L)]
                        pv = plsc.cumsum(freei)
                        eoff = plsc.load_gather(ebuf, [lane * 0 + jj])
                        pos = cnt + eoff + pv - 1
                        take = jnp.logical_and(freei == 1, pos < CAP)
                        plsc.store_scatter(row_i, [pos], ids, mask=take)
                        plsc.store_scatter(assigned, [ids], ones, mask=take)
                    return 0

                lax.fori_loop(0, CHUNK_V // UNROLL, c3, 0)
                return (ci + 1, jnp.minimum(cnt + ctot, CAP))

            lax.while_loop(wcond, wbody, (jnp.int32(0), jnp.int32(0)))
            pltpu.sync_copy(row_i, idx_out.at[e])
            pltpu.sync_copy(row_i, s_out.at[pl.ds(e * CAP, CAP)])

        pltpu.async_copy(s_idx.at[pl.ds(0, T_TOKENS)], cbuf, semA)

        def expert2(i, _):
            e0 = 2 * i
            pltpu.make_async_copy(
                s_idx.at[pl.ds(0, T_TOKENS)], cbuf, semA).wait()
            pltpu.async_copy(
                s_idx.at[pl.ds((e0 + 1) * T_TOKENS, T_TOKENS)], cbuf2, semB)
            process(e0, cbuf)

            pltpu.make_async_copy(
                s_idx.at[pl.ds(0, T_TOKENS)], cbuf2, semB).wait()

            @pl.when(i < NUM_EXPERT // 2 - 1)
            def _pf():
                pltpu.async_copy(
                    s_idx.at[pl.ds((e0 + 2) * T_TOKENS, T_TOKENS)], cbuf,
                    semA)

            process(e0 + 1, cbuf2)
            return 0

        lax.fori_loop(0, NUM_EXPERT // 2, expert2, 0)

    plsc.subcore_barrier()

    @pl.when(on_core0)
    def _emit():
      with jax.named_scope("phase_emit"):
        pltpu.sync_copy(s_out.at[pl.ds(sid * CAP, CAP)], obuf)

        def ev(i, _):
            for u in range(UNROLL):
                o = (i * UNROLL + u) * L
                idsv = obuf[pl.ds(o, L)]
                vrow[pl.ds(o, L)] = plsc.load_gather(col_raw, [idsv])
            return 0

        lax.fori_loop(0, CAP // L // UNROLL, ev, 0)
        pltpu.sync_copy(vrow, val_out.at[sid])


def _assign_sc(affin_t):
    mesh = plsc.VectorSubcoreMesh(core_axis_name="c", subcore_axis_name="s")
    f = functools.partial(
        pl.kernel,
        mesh=mesh,
        compiler_params=pltpu.CompilerParams(needs_layout_passes=False),
        out_type=[
            jax.ShapeDtypeStruct((NUM_EXPERT, CAP), jnp.int32),
            jax.ShapeDtypeStruct((NUM_EXPERT, CAP), jnp.float32),
        ],
        scratch_types=[
            pltpu.VMEM((T_TOKENS,), jnp.float32),
            pltpu.VMEM((T_TOKENS + L,), jnp.int32),
            pltpu.VMEM((T_TOKENS + L,), jnp.int32),
            pltpu.VMEM((T_TOKENS + L,), jnp.int32),
            pltpu.VMEM((T_TOKENS + L,), jnp.int32),
            pltpu.VMEM((NBUCKET * L,), jnp.int32),
            pltpu.VMEM((NBUCKET + L,), jnp.int32),
            pltpu.VMEM((T_TOKENS,), jnp.int32),
            pltpu.VMEM((T_TOKENS,), jnp.int32),
            pltpu.VMEM((T_TOKENS,), jnp.int32),
            pltpu.VMEM((CAP,), jnp.int32),
            pltpu.VMEM((512,), jnp.int32),
            pltpu.VMEM((512,), jnp.int32),
            pltpu.VMEM((32,), jnp.int32),
            pltpu.VMEM((32 + L,), jnp.int32),
            pltpu.VMEM((CAP,), jnp.int32),
            pltpu.VMEM((CAP,), jnp.float32),
            pltpu.SemaphoreType.DMA,
            pltpu.SemaphoreType.DMA,
            pltpu.VMEM_SHARED((NUM_EXPERT * T_TOKENS,), jnp.int32),
            pltpu.VMEM_SHARED((NUM_EXPERT * CAP,), jnp.int32),
        ],
    )(_assign_body)
    return f(affin_t)


def kernel(input_features, expert_centroids):
    features = input_features.reshape(-1, input_features.shape[-1])
    affin_t = _affinities_t(expert_centroids, features)
    top_idx, top_value = _assign_sc(affin_t)
    return top_idx, top_value
